# baseline (speedup 1.0000x reference)
import functools

import jax
import jax.numpy as jnp
from jax import lax
from jax.experimental import pallas as pl
from jax.experimental.pallas import tpu as pltpu
from jax.experimental.pallas import tpu_sc as plsc

_NC = 2
_NS = 16
_NW = _NC * _NS
_C = 384
_NBUF = 2
_T = 168


@functools.lru_cache(maxsize=None)
def _make_sc_kernel(n, d):
    b_per_w = n // _NW
    assert n % _NW == 0
    sizes = [_C] * (b_per_w // _C)
    if b_per_w % _C:
        sizes.append(b_per_w % _C)
    offs = [sum(sizes[:i]) for i in range(len(sizes))]
    nchunks = len(sizes)
    mesh = plsc.VectorSubcoreMesh(core_axis_name="c", subcore_axis_name="s")

    @functools.partial(
        pl.kernel,
        mesh=mesh,
        out_type=jax.ShapeDtypeStruct((n, d), jnp.float32),
        scratch_types=[
            pltpu.VMEM((b_per_w,), jnp.int32),
            pltpu.VMEM((_NBUF, _C, d), jnp.float32),
            pltpu.VMEM((32, d), jnp.float32),
            pltpu.VMEM_SHARED((_T, d), jnp.float32),
        ] + [pltpu.SemaphoreType.DMA] * (2 * _NBUF),
    )
    def sc_kernel(hour_hbm, wk_hbm, idx_hbm, out_hbm,
                  idx_v, rows_v, stage_v, tab_sh, *sems):
        gsem = sems[:_NBUF]
        wsem = sems[_NBUF:]
        sid = lax.axis_index("s")
        wid = sid * _NC + lax.axis_index("c")
        w_base = wid * b_per_w

        pltpu.sync_copy(idx_hbm.at[pl.ds(w_base, b_per_w)], idx_v)

        @pl.when(sid == 0)
        def _():
            pltpu.sync_copy(hour_hbm, stage_v.at[pl.ds(0, 24)])
            pltpu.sync_copy(wk_hbm, stage_v.at[pl.ds(24, 7)])
            build = rows_v.at[0]

            def row(t, carry):
                h = t // 7
                w = t // 24 + 24
                for j in range(d // 16):
                    sl = pl.ds(j * 16, 16)
                    build[t, sl] = stage_v[h, sl] + stage_v[w, sl]
                return carry

            lax.fori_loop(0, _T, row, 0)
            pltpu.sync_copy(build.at[pl.ds(0, _T)], tab_sh)

        plsc.subcore_barrier()

        def start_gather(i):
            b = i % _NBUF
            return pltpu.async_copy(
                tab_sh.at[idx_v.at[pl.ds(offs[i], sizes[i])]],
                rows_v.at[b].at[pl.ds(0, sizes[i])], gsem[b])

        gcp = [None] * _NBUF
        wcp = [None] * _NBUF
        outstanding = [False] * _NBUF
        for i in range(min(_NBUF - 1, nchunks)):
            gcp[i % _NBUF] = start_gather(i)
        for i in range(nchunks):
            b = i % _NBUF
            gcp[b].wait()
            wcp[b] = pltpu.async_copy(
                rows_v.at[b].at[pl.ds(0, sizes[i])],
                out_hbm.at[pl.ds(w_base + offs[i], sizes[i])], wsem[b])
            outstanding[b] = True
            j = i + _NBUF - 1
            if j < nchunks:
                jb = j % _NBUF
                if outstanding[jb]:
                    wcp[jb].wait()
                    outstanding[jb] = False
                gcp[jb] = start_gather(j)
        for b in range(_NBUF):
            if outstanding[b]:
                wcp[b].wait()

    return sc_kernel


def kernel(time_index, hour_embed, weekday_embed):
    B, H = time_index.shape
    D = hour_embed.shape[1]
    N = B * H
    idx = time_index.reshape(N).astype(jnp.int32)
    out = _make_sc_kernel(N, D)(hour_embed, weekday_embed, idx)
    return out.reshape(B, H, D)


# speedup vs baseline: 1.0367x; 1.0367x over previous
import functools

import jax
import jax.numpy as jnp
from jax import lax
from jax.experimental import pallas as pl
from jax.experimental.pallas import tpu as pltpu
from jax.experimental.pallas import tpu_sc as plsc

_NC = 2
_NS = 16
_NW = _NC * _NS
_C = 384
_NBUF = 2
_T = 168


@functools.lru_cache(maxsize=None)
def _make_sc_kernel(n, d):
    b_per_w = n // _NW
    assert n % _NW == 0
    sizes = [_C] * (b_per_w // _C)
    if b_per_w % _C:
        sizes.append(b_per_w % _C)
    offs = [sum(sizes[:i]) for i in range(len(sizes))]
    nchunks = len(sizes)
    mesh = plsc.VectorSubcoreMesh(core_axis_name="c", subcore_axis_name="s")

    @functools.partial(
        pl.kernel,
        mesh=mesh,
        out_type=jax.ShapeDtypeStruct((n, d), jnp.float32),
        scratch_types=[
            pltpu.VMEM((b_per_w,), jnp.int32),
            pltpu.VMEM((_NBUF, _C, d), jnp.float32),
            pltpu.VMEM((32, d), jnp.float32),
            pltpu.VMEM_SHARED((_T, d), jnp.float32),
        ] + [pltpu.SemaphoreType.DMA] * (2 * _NBUF + 1),
    )
    def sc_kernel(hour_hbm, wk_hbm, idx_hbm, out_hbm,
                  idx_v, rows_v, stage_v, tab_sh, *sems):
        gsem = sems[:_NBUF]
        wsem = sems[_NBUF:2 * _NBUF]
        isem = sems[2 * _NBUF]
        sid = lax.axis_index("s")
        wid = sid * _NC + lax.axis_index("c")
        w_base = wid * b_per_w

        icp = pltpu.async_copy(idx_hbm.at[pl.ds(w_base, b_per_w)], idx_v, isem)

        rows_per_builder = _T // 8

        @pl.when(sid < 8)
        def _():
            pltpu.sync_copy(hour_hbm, stage_v.at[pl.ds(0, 24)])
            pltpu.sync_copy(wk_hbm, stage_v.at[pl.ds(24, 7)])
            build = rows_v.at[0]
            t0 = sid * rows_per_builder

            def row(r, carry):
                t = t0 + r
                h = t // 7
                w = t // 24 + 24
                for j in range(d // 16):
                    sl = pl.ds(j * 16, 16)
                    build[r, sl] = stage_v[h, sl] + stage_v[w, sl]
                return carry

            lax.fori_loop(0, rows_per_builder, row, 0)
            pltpu.sync_copy(build.at[pl.ds(0, rows_per_builder)],
                            tab_sh.at[pl.ds(t0, rows_per_builder)])

        plsc.subcore_barrier()
        icp.wait()

        def start_gather(i):
            b = i % _NBUF
            return pltpu.async_copy(
                tab_sh.at[idx_v.at[pl.ds(offs[i], sizes[i])]],
                rows_v.at[b].at[pl.ds(0, sizes[i])], gsem[b])

        gcp = [None] * _NBUF
        wcp = [None] * _NBUF
        outstanding = [False] * _NBUF
        for i in range(min(_NBUF - 1, nchunks)):
            gcp[i % _NBUF] = start_gather(i)
        for i in range(nchunks):
            b = i % _NBUF
            gcp[b].wait()
            wcp[b] = pltpu.async_copy(
                rows_v.at[b].at[pl.ds(0, sizes[i])],
                out_hbm.at[pl.ds(w_base + offs[i], sizes[i])], wsem[b])
            outstanding[b] = True
            j = i + _NBUF - 1
            if j < nchunks:
                jb = j % _NBUF
                if outstanding[jb]:
                    wcp[jb].wait()
                    outstanding[jb] = False
                gcp[jb] = start_gather(j)
        for b in range(_NBUF):
            if outstanding[b]:
                wcp[b].wait()

    return sc_kernel


def kernel(time_index, hour_embed, weekday_embed):
    B, H = time_index.shape
    D = hour_embed.shape[1]
    N = B * H
    idx = time_index.reshape(N).astype(jnp.int32)
    out = _make_sc_kernel(N, D)(hour_embed, weekday_embed, idx)
    return out.reshape(B, H, D)
